# Initial kernel scaffold; baseline (speedup 1.0000x reference)
import functools

import jax
import jax.numpy as jnp
from jax import lax
from jax.experimental import pallas as pl
from jax.experimental.pallas import tpu as pltpu
from jax.experimental.pallas import tpu_sc as plsc

N = 10000
E_IN = 256
HID = 256
OUT = 128
N_EDGES = 160000
FF = 2048
N_LAYERS = 2

NC = 2
NS = 16
RPT = N // NS
DEG_CHUNK = 40
DEG_NCHUNK = (N_EDGES // (NC * NS)) // DEG_CHUNK
ROW_CHUNK = 80
ROW_NCHUNK = (N_EDGES // NS) // ROW_CHUNK

_mesh = plsc.VectorSubcoreMesh(
    core_axis_name="c", subcore_axis_name="s", num_cores=NC, num_subcores=NS)


@functools.partial(
    pl.kernel,
    out_type=jax.ShapeDtypeStruct((NC, N, 16), jnp.float32),
    mesh=_mesh,
    scratch_types=[
        pltpu.VMEM((DEG_NCHUNK, DEG_CHUNK), jnp.int32),
        pltpu.VMEM((DEG_CHUNK, 16), jnp.float32),
        pltpu.VMEM((RPT, 16), jnp.float32),
        pltpu.VMEM_SHARED((N, 16), jnp.float32),
    ],
)
def _deg_kernel(dst3_hbm, out_hbm, dst_idx, ones_rows, wb, hist_s):
    c = lax.axis_index("c")
    s = lax.axis_index("s")
    z = jnp.zeros((16,), jnp.float32)
    lane = lax.iota(jnp.int32, 16)
    one_row = jnp.where(lane == 0, 1.0, 0.0).astype(jnp.float32)

    def init_ones(i, carry):
        ones_rows[i] = one_row
        return carry
    lax.fori_loop(0, DEG_CHUNK, init_ones, 0)

    def zero_wb(i, carry):
        wb[i] = z
        return carry
    lax.fori_loop(0, RPT, zero_wb, 0)

    sync = pltpu.sync_copy
    sync(wb, hist_s.at[pl.ds(s * RPT, RPT)])
    sync(dst3_hbm.at[c, s], dst_idx)
    plsc.subcore_barrier()

    def scat(j, carry):
        sync(ones_rows, hist_s.at[dst_idx.at[j]], add=True)
        return carry
    lax.fori_loop(0, DEG_NCHUNK, scat, 0)
    plsc.subcore_barrier()

    sync(hist_s.at[pl.ds(s * RPT, RPT)], wb)
    sync(wb, out_hbm.at[c, pl.ds(s * RPT, RPT)])


@functools.partial(
    pl.kernel,
    out_type=jax.ShapeDtypeStruct((NC, N, 128), jnp.float32),
    mesh=_mesh,
    scratch_types=[
        pltpu.VMEM((ROW_NCHUNK, ROW_CHUNK), jnp.int32),
        pltpu.VMEM((ROW_NCHUNK, ROW_CHUNK), jnp.int32),
        pltpu.VMEM((ROW_CHUNK, 128), jnp.float32),
        pltpu.VMEM((125, 128), jnp.float32),
        pltpu.VMEM_SHARED((N, 128), jnp.float32),
    ],
)
def _row_scatter_kernel(g_hbm, src3_hbm, dst3_hbm, out_hbm,
                        src_idx, dst_idx, rows, wb, acc):
    c = lax.axis_index("c")
    s = lax.axis_index("s")
    z = jnp.zeros((16,), jnp.float32)
    sync = pltpu.sync_copy

    def zero_wb(i, carry):
        for k in range(8):
            wb[i, pl.ds(k * 16, 16)] = z
        return carry
    lax.fori_loop(0, 125, zero_wb, 0)

    def zero_acc(k, carry):
        sync(wb, acc.at[pl.ds(s * RPT + k * 125, 125)])
        return carry
    lax.fori_loop(0, 5, zero_acc, 0)

    sync(src3_hbm.at[c, s], src_idx)
    sync(dst3_hbm.at[s], dst_idx)
    plsc.subcore_barrier()

    def scat(j, carry):
        sync(g_hbm.at[src_idx.at[j]], rows)
        sync(rows, acc.at[dst_idx.at[j]], add=True)
        return carry
    lax.fori_loop(0, ROW_NCHUNK, scat, 0)
    plsc.subcore_barrier()

    def writeback(k, carry):
        sync(acc.at[pl.ds(s * RPT + k * 125, 125)], wb)
        sync(wb, out_hbm.at[c, pl.ds(s * RPT + k * 125, 125)])
        return carry
    lax.fori_loop(0, 5, writeback, 0)


def _dinv_from_hist(hist_ref):
    deg = (jnp.sum(hist_ref[0], axis=-1, keepdims=True)
           + jnp.sum(hist_ref[1], axis=-1, keepdims=True) + 1.0)
    return lax.rsqrt(deg)


def _mm(a, w):
    return lax.dot_general(a, w, (((1,), (0,)), ((), ())),
                           preferred_element_type=jnp.float32)


def _mmT(a, w):
    return lax.dot_general(a, w, (((1,), (1,)), ((), ())),
                           preferred_element_type=jnp.float32)


def _k1_body(x_ref, w1_ref, hist_ref, out_ref):
    dinv = _dinv_from_hist(hist_ref)
    g = _mm(x_ref[...], w1_ref[...]) * dinv
    out_ref[0] = g[:, :128]
    out_ref[1] = g[:, 128:]


def _k2_body(t_ref, g_ref, hist_ref, w2_ref, b1_ref, out_ref):
    dinv = _dinv_from_hist(hist_ref)
    t = jnp.concatenate([t_ref[0], t_ref[1]], axis=-1)
    g = jnp.concatenate([g_ref[0], g_ref[1]], axis=-1)
    u = jnp.maximum(dinv * (t + g) + b1_ref[...], 0.0)
    g2 = _mm(u, w2_ref[...]) * dinv
    out_ref[0] = g2[:, :128]
    out_ref[1] = g2[:, 128:]


def _layer_norm(h, g, b):
    m = jnp.mean(h, axis=-1, keepdims=True)
    v = jnp.mean((h - m) ** 2, axis=-1, keepdims=True)
    return (h - m) * lax.rsqrt(v + 1e-5) * g + b


def _k3_body(t_ref, g_ref, hist_ref, b2_ref, wfc_ref, bfc_ref, *refs):
    lrefs, out_ref = refs[:-1], refs[-1]
    dinv = _dinv_from_hist(hist_ref)
    t = jnp.concatenate([t_ref[0], t_ref[1]], axis=-1)
    g = jnp.concatenate([g_ref[0], g_ref[1]], axis=-1)
    h = dinv * (t + g) + b2_ref[...]
    for l in range(N_LAYERS):
        (wv, bv, wo, bo, ln1g, ln1b, wf1, bf1, wf2, bf2, ln2g, ln2b) = (
            lrefs[12 * l + i] for i in range(12))
        a = _mmT(_mmT(h, wv[...]) + bv[...], wo[...]) + bo[...]
        h = _layer_norm(h + a, ln1g[...], ln1b[...])
        f = _mmT(jnp.maximum(_mmT(h, wf1[...]) + bf1[...], 0.0), wf2[...]) \
            + bf2[...]
        h = _layer_norm(h + f, ln2g[...], ln2b[...])
    out_ref[...] = _mmT(h, wfc_ref[...]) + bfc_ref[...]


def _full(shape):
    return pl.BlockSpec(shape, lambda i: (0,) * len(shape))


def _split_spec(t):
    return pl.BlockSpec((2, t, 128), lambda i: (0, i, 0))


def _hist_spec(t):
    return pl.BlockSpec((2, t, 16), lambda i: (0, i, 0))


def kernel(x, params, edge_index):
    p = params
    src = edge_index[0].astype(jnp.int32)
    dst = edge_index[1].astype(jnp.int32)

    dst3_deg = dst.reshape(NC, NS, DEG_NCHUNK, DEG_CHUNK)
    src3 = jnp.stack([src, src + N]).reshape(NC, NS, ROW_NCHUNK, ROW_CHUNK)
    dst3 = dst.reshape(NS, ROW_NCHUNK, ROW_CHUNK)

    hist = _deg_kernel(dst3_deg)

    T1 = 2000
    g1 = pl.pallas_call(
        _k1_body,
        grid=(N // T1,),
        in_specs=[pl.BlockSpec((T1, E_IN), lambda i: (i, 0)),
                  _full((E_IN, HID)),
                  _hist_spec(T1)],
        out_specs=_split_spec(T1),
        out_shape=jax.ShapeDtypeStruct((2, N, 128), jnp.float32),
    )(x, p["W1"], hist)

    t1 = _row_scatter_kernel(g1.reshape(2 * N, 128), src3, dst3)

    g2 = pl.pallas_call(
        _k2_body,
        grid=(N // T1,),
        in_specs=[_split_spec(T1), _split_spec(T1), _hist_spec(T1),
                  _full((HID, HID)), _full((1, HID))],
        out_specs=_split_spec(T1),
        out_shape=jax.ShapeDtypeStruct((2, N, 128), jnp.float32),
    )(t1, g1, hist, p["W2"], p["b1"].reshape(1, HID))

    t2 = _row_scatter_kernel(g2.reshape(2 * N, 128), src3, dst3)

    layer_args = []
    layer_specs = []
    for lp in p["layers"]:
        wv = lp["Wqkv"][2 * HID:]
        bv = lp["bqkv"][2 * HID:].reshape(1, HID)
        layer_args += [
            wv, bv, lp["Wo"], lp["bo"].reshape(1, HID),
            lp["ln1_g"].reshape(1, HID), lp["ln1_b"].reshape(1, HID),
            lp["Wff1"], lp["bff1"].reshape(1, FF),
            lp["Wff2"], lp["bff2"].reshape(1, HID),
            lp["ln2_g"].reshape(1, HID), lp["ln2_b"].reshape(1, HID),
        ]
        layer_specs += [
            _full((HID, HID)), _full((1, HID)),
            _full((HID, HID)), _full((1, HID)),
            _full((1, HID)), _full((1, HID)),
            _full((FF, HID)), _full((1, FF)),
            _full((HID, FF)), _full((1, HID)),
            _full((1, HID)), _full((1, HID)),
        ]

    T3 = 1000
    out = pl.pallas_call(
        _k3_body,
        grid=(N // T3,),
        in_specs=[_split_spec(T3), _split_spec(T3), _hist_spec(T3),
                  _full((1, HID)), _full((OUT, HID)), _full((1, OUT))]
                 + layer_specs,
        out_specs=pl.BlockSpec((T3, OUT), lambda i: (i, 0)),
        out_shape=jax.ShapeDtypeStruct((N, OUT), jnp.float32),
    )(t2, g2, hist, p["b2"].reshape(1, HID), p["Wfc"],
      p["bfc"].reshape(1, OUT), *layer_args)

    return out


# speedup vs baseline: 6.3647x; 6.3647x over previous
import functools

import jax
import jax.numpy as jnp
from jax import lax
from jax.experimental import pallas as pl
from jax.experimental.pallas import tpu as pltpu
from jax.experimental.pallas import tpu_sc as plsc

N = 10000
E_IN = 256
HID = 256
OUT = 128
N_EDGES = 160000
FF = 2048
N_LAYERS = 2

NC = 2
NS = 16
NP = 10240
PH = NP // 2
DUM = 256
ACCR = PH + DUM
APT = ACCR // NS
RPP = PH // NS
DEG_CHUNK = 40
DEG_NCHUNK = (N_EDGES // (NC * NS)) // DEG_CHUNK
ROW_CHUNK = 80
ROW_NCHUNK = (N_EDGES // NS) // ROW_CHUNK

def _zero_wb(wb, nrows, ncolv):
    z = jnp.zeros((16,), jnp.float32)

    def body(i, carry):
        for k in range(ncolv):
            wb[i, pl.ds(k * 16, 16)] = z
        return carry
    lax.fori_loop(0, nrows, body, 0)


def _phase(s, p, acc, wb, out_hbm, out_base, scat_loop):
    sync = pltpu.sync_copy

    def zero_acc(k, carry):
        sync(wb.at[pl.ds(0, 168)], acc.at[pl.ds(s * APT + k * 168, 168)])
        return carry
    lax.fori_loop(0, 2, zero_acc, 0)
    plsc.subcore_barrier()

    scat_loop()
    plsc.subcore_barrier()

    def writeback(k, carry):
        sync(acc.at[pl.ds(s * RPP + k * 160, 160)],
             out_hbm.at[out_base, pl.ds(p * PH + s * RPP + k * 160, 160)])
        return carry
    lax.fori_loop(0, 2, writeback, 0)
    plsc.subcore_barrier()


def _deg_body(dst4_hbm, out_hbm, dst_idx, ones_rows, wb, hist_s):
    c = lax.axis_index("c")
    s = lax.axis_index("s")
    sync = pltpu.sync_copy
    one = jnp.ones((16,), jnp.float32)

    def init_ones(i, carry):
        for k in range(8):
            ones_rows[i, pl.ds(k * 16, 16)] = one
        return carry
    lax.fori_loop(0, DEG_CHUNK, init_ones, 0)
    _zero_wb(wb, 168, 8)

    for p in range(2):
        sync(dst4_hbm.at[p, c, s], dst_idx)

        def scat():
            def body(j, carry):
                sync(ones_rows, hist_s.at[dst_idx.at[j]], add=True)
                return carry
            lax.fori_loop(0, DEG_NCHUNK, body, 0)
        _phase(s, p, hist_s, wb, out_hbm, c, scat)


def _row_body(g_hbm, src2_hbm, dst4_hbm, out_hbm,
              src_idx, dst_idx, rows, wb, acc):
    c = lax.axis_index("c")
    s = lax.axis_index("s")
    sync = pltpu.sync_copy

    _zero_wb(wb, 168, 8)
    sync(src2_hbm.at[c, s], src_idx)

    for p in range(2):
        sync(dst4_hbm.at[p, s], dst_idx)

        def scat():
            def body(j, carry):
                sync(g_hbm.at[src_idx.at[j]], rows)
                sync(rows, acc.at[dst_idx.at[j]], add=True)
                return carry
            lax.fori_loop(0, ROW_NCHUNK, body, 0)
        _phase(s, p, acc, wb, out_hbm, c, scat)


@functools.cache
def _sc_kernels():
    mesh = plsc.VectorSubcoreMesh(
        core_axis_name="c", subcore_axis_name="s",
        num_cores=NC, num_subcores=NS)
    deg = pl.kernel(
        _deg_body,
        out_type=jax.ShapeDtypeStruct((NC, NP, 128), jnp.float32),
        mesh=mesh,
        scratch_types=[
            pltpu.VMEM((DEG_NCHUNK, DEG_CHUNK), jnp.int32),
            pltpu.VMEM((DEG_CHUNK, 128), jnp.float32),
            pltpu.VMEM((168, 128), jnp.float32),
            pltpu.VMEM_SHARED((ACCR, 128), jnp.float32),
        ],
    )
    row = pl.kernel(
        _row_body,
        out_type=jax.ShapeDtypeStruct((NC, NP, 128), jnp.float32),
        mesh=mesh,
        scratch_types=[
            pltpu.VMEM((ROW_NCHUNK, ROW_CHUNK), jnp.int32),
            pltpu.VMEM((ROW_NCHUNK, ROW_CHUNK), jnp.int32),
            pltpu.VMEM((ROW_CHUNK, 128), jnp.float32),
            pltpu.VMEM((168, 128), jnp.float32),
            pltpu.VMEM_SHARED((ACCR, 128), jnp.float32),
        ],
    )
    return deg, row


def _dinv_from_hist(hist_ref):
    deg = hist_ref[0][:, 0:1] + hist_ref[1][:, 0:1] + 1.0
    return lax.rsqrt(deg)


def _mm(a, w):
    return lax.dot_general(a, w, (((1,), (0,)), ((), ())),
                           preferred_element_type=jnp.float32)


def _mmT(a, w):
    return lax.dot_general(a, w, (((1,), (1,)), ((), ())),
                           preferred_element_type=jnp.float32)


def _split_out(out_ref, g):
    out_ref[0] = g[:, :128]
    out_ref[1] = g[:, 128:]


def _cat2(ref):
    return jnp.concatenate([ref[0], ref[1]], axis=-1)


def _cat(ref):
    return jnp.concatenate([ref[qq] for qq in range(4)], axis=-1)


def _k1_body(x_ref, w1_ref, hist_ref, out_ref):
    dinv = _dinv_from_hist(hist_ref)
    g = _mm(x_ref[...], w1_ref[...]) * dinv
    _split_out(out_ref, g)


def _k2_body(t_ref, g_ref, hist_ref, w2_ref, b1_ref, out_ref):
    dinv = _dinv_from_hist(hist_ref)
    u = jnp.maximum(dinv * (_cat2(t_ref) + _cat2(g_ref)) + b1_ref[...], 0.0)
    g2 = _mm(u, w2_ref[...]) * dinv
    _split_out(out_ref, g2)


def _layer_norm(h, g, b):
    m = jnp.mean(h, axis=-1, keepdims=True)
    v = jnp.mean((h - m) ** 2, axis=-1, keepdims=True)
    return (h - m) * lax.rsqrt(v + 1e-5) * g + b


def _k3_body(t_ref, g_ref, hist_ref, b2_ref, wfc_ref, bfc_ref, *refs):
    lrefs, out_ref = refs[:-1], refs[-1]
    dinv = _dinv_from_hist(hist_ref)
    h = dinv * (_cat2(t_ref) + _cat2(g_ref)) + b2_ref[...]
    for l in range(N_LAYERS):
        (wv, bv, wo, bo, ln1g, ln1b, wf1, bf1, wf2, bf2, ln2g, ln2b) = (
            lrefs[12 * l + i] for i in range(12))
        a = _mmT(_mmT(h, wv[...]) + bv[...], wo[...]) + bo[...]
        h = _layer_norm(h + a, ln1g[...], ln1b[...])
        f = _mmT(jnp.maximum(_mmT(h, wf1[...]) + bf1[...], 0.0), wf2[...]) \
            + bf2[...]
        h = _layer_norm(h + f, ln2g[...], ln2b[...])
    out_ref[...] = _mmT(h, wfc_ref[...]) + bfc_ref[...]


def _full(shape):
    return pl.BlockSpec(shape, lambda i: (0,) * len(shape))


def _half_spec(t):
    return pl.BlockSpec((2, t, 128), lambda i: (0, i, 0))


def kernel(x, params, edge_index):
    p = params
    src = edge_index[0].astype(jnp.int32)
    dst = edge_index[1].astype(jnp.int32)

    dum = PH + (dst & (DUM - 1))
    ph0 = jnp.where(dst < PH, dst, dum)
    ph1 = jnp.where(dst >= PH, dst - PH, dum)
    dst_ph = jnp.stack([ph0, ph1])

    src2 = jnp.stack([src, src + N]) \
        .reshape(NC, NS, ROW_NCHUNK, ROW_CHUNK)
    dst4_row = dst_ph.reshape(2, NS, ROW_NCHUNK, ROW_CHUNK)
    dst4_deg = dst_ph.reshape(2, NC, NS, DEG_NCHUNK, DEG_CHUNK)

    deg_kernel, row_scatter_kernel = _sc_kernels()
    hist = deg_kernel(dst4_deg)

    T1 = 2000
    g1 = pl.pallas_call(
        _k1_body,
        grid=(N // T1,),
        in_specs=[pl.BlockSpec((T1, E_IN), lambda i: (i, 0)),
                  _full((E_IN, HID)),
                  _half_spec(T1)],
        out_specs=_half_spec(T1),
        out_shape=jax.ShapeDtypeStruct((2, N, 128), jnp.float32),
    )(x, p["W1"], hist)

    t1 = row_scatter_kernel(g1.reshape(2 * N, 128), src2, dst4_row)

    g2 = pl.pallas_call(
        _k2_body,
        grid=(N // T1,),
        in_specs=[_half_spec(T1), _half_spec(T1), _half_spec(T1),
                  _full((HID, HID)), _full((1, HID))],
        out_specs=_half_spec(T1),
        out_shape=jax.ShapeDtypeStruct((2, N, 128), jnp.float32),
    )(t1, g1, hist, p["W2"], p["b1"].reshape(1, HID))

    t2 = row_scatter_kernel(g2.reshape(2 * N, 128), src2, dst4_row)

    layer_args = []
    layer_specs = []
    for lp in p["layers"]:
        wv = lp["Wqkv"][2 * HID:]
        bv = lp["bqkv"][2 * HID:].reshape(1, HID)
        layer_args += [
            wv, bv, lp["Wo"], lp["bo"].reshape(1, HID),
            lp["ln1_g"].reshape(1, HID), lp["ln1_b"].reshape(1, HID),
            lp["Wff1"], lp["bff1"].reshape(1, FF),
            lp["Wff2"], lp["bff2"].reshape(1, HID),
            lp["ln2_g"].reshape(1, HID), lp["ln2_b"].reshape(1, HID),
        ]
        layer_specs += [
            _full((HID, HID)), _full((1, HID)),
            _full((HID, HID)), _full((1, HID)),
            _full((1, HID)), _full((1, HID)),
            _full((FF, HID)), _full((1, FF)),
            _full((HID, FF)), _full((1, HID)),
            _full((1, HID)), _full((1, HID)),
        ]

    T3 = 1000
    out = pl.pallas_call(
        _k3_body,
        grid=(N // T3,),
        in_specs=[_half_spec(T3), _half_spec(T3), _half_spec(T3),
                  _full((1, HID)), _full((OUT, HID)), _full((1, OUT))]
                 + layer_specs,
        out_specs=pl.BlockSpec((T3, OUT), lambda i: (i, 0)),
        out_shape=jax.ShapeDtypeStruct((N, OUT), jnp.float32),
    )(t2, g2, hist, p["b2"].reshape(1, HID), p["Wfc"],
      p["bfc"].reshape(1, OUT), *layer_args)

    return out


# speedup vs baseline: 10.3098x; 1.6199x over previous
import functools

import jax
import jax.numpy as jnp
from jax import lax
from jax.experimental import pallas as pl
from jax.experimental.pallas import tpu as pltpu
from jax.experimental.pallas import tpu_sc as plsc

N = 10000
E_IN = 256
HID = 256
OUT = 128
N_EDGES = 160000
FF = 2048
N_LAYERS = 2

NC = 2
NS = 16
NP = 10240
PH = NP // 2
DUM = 256
ACCR = PH + DUM
APT = ACCR // NS
RPP = PH // NS
DEG_CHUNK = 40
DEG_NCHUNK = (N_EDGES // (NC * NS)) // DEG_CHUNK
ROW_CHUNK = 100
ROW_NCHUNK = (N_EDGES // NS) // ROW_CHUNK

def _zero_wb(wb, nrows, ncolv):
    z = jnp.zeros((16,), jnp.float32)

    def body(i, carry):
        for k in range(ncolv):
            wb[i, pl.ds(k * 16, 16)] = z
        return carry
    lax.fori_loop(0, nrows, body, 0)


def _phase(s, p, acc, wb, out_hbm, out_base, scat_loop):
    sync = pltpu.sync_copy

    def zero_acc(k, carry):
        sync(wb.at[pl.ds(0, 168)], acc.at[pl.ds(s * APT + k * 168, 168)])
        return carry
    lax.fori_loop(0, 2, zero_acc, 0)
    plsc.subcore_barrier()

    scat_loop()
    plsc.subcore_barrier()

    def writeback(k, carry):
        sync(acc.at[pl.ds(s * RPP + k * 160, 160)],
             out_hbm.at[out_base, pl.ds(p * PH + s * RPP + k * 160, 160)])
        return carry
    lax.fori_loop(0, 2, writeback, 0)
    plsc.subcore_barrier()


def _deg_body(dst4_hbm, out_hbm, dst_idx, ones_rows, wb, hist_s):
    c = lax.axis_index("c")
    s = lax.axis_index("s")
    sync = pltpu.sync_copy
    one = jnp.ones((16,), jnp.float32)

    def init_ones(i, carry):
        for k in range(8):
            ones_rows[i, pl.ds(k * 16, 16)] = one
        return carry
    lax.fori_loop(0, DEG_CHUNK, init_ones, 0)
    _zero_wb(wb, 168, 8)

    for p in range(2):
        sync(dst4_hbm.at[p, c, s], dst_idx)

        def scat():
            def body(j, carry):
                sync(ones_rows, hist_s.at[dst_idx.at[j]], add=True)
                return carry
            lax.fori_loop(0, DEG_NCHUNK, body, 0)
        _phase(s, p, hist_s, wb, out_hbm, c, scat)


def _row_body(g_hbm, src2_hbm, dst4_hbm, out_hbm,
              src_idx, dst_idx, rows0, rows1, wb, acc, sem0, sem1):
    c = lax.axis_index("c")
    s = lax.axis_index("s")
    sync = pltpu.sync_copy

    _zero_wb(wb, 168, 8)
    sync(src2_hbm.at[c, s], src_idx)

    for p in range(2):
        sync(dst4_hbm.at[p, s], dst_idx)

        def scat():
            pltpu.async_copy(g_hbm.at[src_idx.at[0]], rows0, sem0)
            pltpu.async_copy(g_hbm.at[src_idx.at[1]], rows1, sem1)

            def body(jj, carry):
                j0 = 2 * jj
                pltpu.make_async_copy(
                    g_hbm.at[src_idx.at[j0]], rows0, sem0).wait()
                sync(rows0, acc.at[dst_idx.at[j0]], add=True)
                pltpu.async_copy(g_hbm.at[src_idx.at[j0 + 2]], rows0, sem0)
                pltpu.make_async_copy(
                    g_hbm.at[src_idx.at[j0 + 1]], rows1, sem1).wait()
                sync(rows1, acc.at[dst_idx.at[j0 + 1]], add=True)
                pltpu.async_copy(g_hbm.at[src_idx.at[j0 + 3]], rows1, sem1)
                return carry
            lax.fori_loop(0, ROW_NCHUNK // 2 - 1, body, 0)

            last = ROW_NCHUNK - 2
            pltpu.make_async_copy(
                g_hbm.at[src_idx.at[last]], rows0, sem0).wait()
            sync(rows0, acc.at[dst_idx.at[last]], add=True)
            pltpu.make_async_copy(
                g_hbm.at[src_idx.at[last + 1]], rows1, sem1).wait()
            sync(rows1, acc.at[dst_idx.at[last + 1]], add=True)
        _phase(s, p, acc, wb, out_hbm, c, scat)


@functools.cache
def _sc_kernels():
    mesh = plsc.VectorSubcoreMesh(
        core_axis_name="c", subcore_axis_name="s",
        num_cores=NC, num_subcores=NS)
    deg = pl.kernel(
        _deg_body,
        out_type=jax.ShapeDtypeStruct((NC, NP, 128), jnp.float32),
        mesh=mesh,
        scratch_types=[
            pltpu.VMEM((DEG_NCHUNK, DEG_CHUNK), jnp.int32),
            pltpu.VMEM((DEG_CHUNK, 128), jnp.float32),
            pltpu.VMEM((168, 128), jnp.float32),
            pltpu.VMEM_SHARED((ACCR, 128), jnp.float32),
        ],
    )
    row = pl.kernel(
        _row_body,
        out_type=jax.ShapeDtypeStruct((NC, NP, 128), jnp.float32),
        mesh=mesh,
        scratch_types=[
            pltpu.VMEM((ROW_NCHUNK, ROW_CHUNK), jnp.int32),
            pltpu.VMEM((ROW_NCHUNK, ROW_CHUNK), jnp.int32),
            pltpu.VMEM((ROW_CHUNK, 128), jnp.float32),
            pltpu.VMEM((ROW_CHUNK, 128), jnp.float32),
            pltpu.VMEM((168, 128), jnp.float32),
            pltpu.VMEM_SHARED((ACCR, 128), jnp.float32),
            pltpu.SemaphoreType.DMA,
            pltpu.SemaphoreType.DMA,
        ],
    )
    return deg, row


def _dinv_from_hist(hist_ref):
    deg = hist_ref[0][:, 0:1] + hist_ref[1][:, 0:1] + 1.0
    return lax.rsqrt(deg)


def _mm(a, w):
    return lax.dot_general(a, w, (((1,), (0,)), ((), ())),
                           preferred_element_type=jnp.float32)


def _mmT(a, w):
    return lax.dot_general(a, w, (((1,), (1,)), ((), ())),
                           preferred_element_type=jnp.float32)


def _split_out(out_ref, g):
    out_ref[0] = g[:, :128]
    out_ref[1] = g[:, 128:]


def _cat2(ref):
    return jnp.concatenate([ref[0], ref[1]], axis=-1)


def _cat(ref):
    return jnp.concatenate([ref[qq] for qq in range(4)], axis=-1)


def _k1_body(x_ref, w1_ref, hist_ref, out_ref):
    dinv = _dinv_from_hist(hist_ref)
    g = _mm(x_ref[...], w1_ref[...]) * dinv
    _split_out(out_ref, g)


def _k2_body(t_ref, g_ref, hist_ref, w2_ref, b1_ref, out_ref):
    dinv = _dinv_from_hist(hist_ref)
    u = jnp.maximum(dinv * (_cat2(t_ref) + _cat2(g_ref)) + b1_ref[...], 0.0)
    g2 = _mm(u, w2_ref[...]) * dinv
    _split_out(out_ref, g2)


def _layer_norm(h, g, b):
    m = jnp.mean(h, axis=-1, keepdims=True)
    v = jnp.mean((h - m) ** 2, axis=-1, keepdims=True)
    return (h - m) * lax.rsqrt(v + 1e-5) * g + b


def _k3_body(t_ref, g_ref, hist_ref, b2_ref, wfc_ref, bfc_ref, *refs):
    lrefs, out_ref = refs[:-1], refs[-1]
    dinv = _dinv_from_hist(hist_ref)
    h = dinv * (_cat2(t_ref) + _cat2(g_ref)) + b2_ref[...]
    for l in range(N_LAYERS):
        (wv, bv, wo, bo, ln1g, ln1b, wf1, bf1, wf2, bf2, ln2g, ln2b) = (
            lrefs[12 * l + i] for i in range(12))
        a = _mmT(_mmT(h, wv[...]) + bv[...], wo[...]) + bo[...]
        h = _layer_norm(h + a, ln1g[...], ln1b[...])
        f = _mmT(jnp.maximum(_mmT(h, wf1[...]) + bf1[...], 0.0), wf2[...]) \
            + bf2[...]
        h = _layer_norm(h + f, ln2g[...], ln2b[...])
    out_ref[...] = _mmT(h, wfc_ref[...]) + bfc_ref[...]


def _full(shape):
    return pl.BlockSpec(shape, lambda i: (0,) * len(shape))


def _half_spec(t):
    return pl.BlockSpec((2, t, 128), lambda i: (0, i, 0))


def kernel(x, params, edge_index):
    p = params
    src = edge_index[0].astype(jnp.int32)
    dst = edge_index[1].astype(jnp.int32)

    dum = PH + (dst & (DUM - 1))
    ph0 = jnp.where(dst < PH, dst, dum)
    ph1 = jnp.where(dst >= PH, dst - PH, dum)
    dst_ph = jnp.stack([ph0, ph1])

    src2 = jnp.stack([src, src + N]) \
        .reshape(NC, NS, ROW_NCHUNK, ROW_CHUNK)
    dst4_row = dst_ph.reshape(2, NS, ROW_NCHUNK, ROW_CHUNK)
    dst4_deg = dst_ph.reshape(2, NC, NS, DEG_NCHUNK, DEG_CHUNK)

    deg_kernel, row_scatter_kernel = _sc_kernels()
    hist = deg_kernel(dst4_deg)

    T1 = 2000
    g1 = pl.pallas_call(
        _k1_body,
        grid=(N // T1,),
        in_specs=[pl.BlockSpec((T1, E_IN), lambda i: (i, 0)),
                  _full((E_IN, HID)),
                  _half_spec(T1)],
        out_specs=_half_spec(T1),
        out_shape=jax.ShapeDtypeStruct((2, N, 128), jnp.float32),
    )(x, p["W1"], hist)

    t1 = row_scatter_kernel(g1.reshape(2 * N, 128), src2, dst4_row)

    g2 = pl.pallas_call(
        _k2_body,
        grid=(N // T1,),
        in_specs=[_half_spec(T1), _half_spec(T1), _half_spec(T1),
                  _full((HID, HID)), _full((1, HID))],
        out_specs=_half_spec(T1),
        out_shape=jax.ShapeDtypeStruct((2, N, 128), jnp.float32),
    )(t1, g1, hist, p["W2"], p["b1"].reshape(1, HID))

    t2 = row_scatter_kernel(g2.reshape(2 * N, 128), src2, dst4_row)

    layer_args = []
    layer_specs = []
    for lp in p["layers"]:
        wv = lp["Wqkv"][2 * HID:]
        bv = lp["bqkv"][2 * HID:].reshape(1, HID)
        layer_args += [
            wv, bv, lp["Wo"], lp["bo"].reshape(1, HID),
            lp["ln1_g"].reshape(1, HID), lp["ln1_b"].reshape(1, HID),
            lp["Wff1"], lp["bff1"].reshape(1, FF),
            lp["Wff2"], lp["bff2"].reshape(1, HID),
            lp["ln2_g"].reshape(1, HID), lp["ln2_b"].reshape(1, HID),
        ]
        layer_specs += [
            _full((HID, HID)), _full((1, HID)),
            _full((HID, HID)), _full((1, HID)),
            _full((1, HID)), _full((1, HID)),
            _full((FF, HID)), _full((1, FF)),
            _full((HID, FF)), _full((1, HID)),
            _full((1, HID)), _full((1, HID)),
        ]

    T3 = 1000
    out = pl.pallas_call(
        _k3_body,
        grid=(N // T3,),
        in_specs=[_half_spec(T3), _half_spec(T3), _half_spec(T3),
                  _full((1, HID)), _full((OUT, HID)), _full((1, OUT))]
                 + layer_specs,
        out_specs=pl.BlockSpec((T3, OUT), lambda i: (i, 0)),
        out_shape=jax.ShapeDtypeStruct((N, OUT), jnp.float32),
    )(t2, g2, hist, p["b2"].reshape(1, HID), p["Wfc"],
      p["bfc"].reshape(1, OUT), *layer_args)

    return out
